# Initial kernel scaffold; baseline (speedup 1.0000x reference)
import jax
import jax.numpy as jnp
from jax.experimental import pallas as pl


def kernel(e, pt, eta, phi, m, edge_index, W1, b1, W2, b2):
    raise NotImplementedError("write your pallas kernel here")


# speedup vs baseline: 24.7557x; 24.7557x over previous
import functools

import jax
import jax.numpy as jnp
from jax import lax
from jax.experimental import pallas as pl
from jax.experimental.pallas import tpu as pltpu
from jax.experimental.pallas import tpu_sc as plsc

N = 10000
E = 160000
NR = 10240
F = 8
NSC = 2
NTPC = 16
NTILES = NSC * NTPC
CHUNK = 128
NCHUNK = 40
EPT = NCHUNK * CHUNK
EP = NTILES * EPT
TRASH = N
RPT = NR // NTPC
NRR = NR * F // 128

def _sc_body(u_hbm, src_hbm, dst_hbm, zeros_hbm, out_hbm, srcv, dstv, buf, sem, shared):
    c = lax.axis_index("c")
    s = lax.axis_index("s")
    wid = c * NTPC + s
    row0 = s * RPT
    pltpu.sync_copy(zeros_hbm.at[pl.ds(row0, RPT)], shared.at[pl.ds(row0, RPT)])
    pltpu.sync_copy(src_hbm.at[wid], srcv)
    pltpu.sync_copy(dst_hbm.at[wid], dstv)
    plsc.subcore_barrier()

    @pl.loop(0, NCHUNK)
    def _chunk(j):
        pltpu.async_copy(u_hbm.at[srcv.at[j]], buf, sem).wait()
        pltpu.sync_copy(buf, shared.at[dstv.at[j]], add=True)

    plsc.subcore_barrier()
    pltpu.sync_copy(shared.at[pl.ds(row0, RPT)], out_hbm.at[c, pl.ds(row0, RPT)])


@functools.cache
def _get_sc_agg():
    mesh = plsc.VectorSubcoreMesh(core_axis_name="c", subcore_axis_name="s",
                                  num_cores=NSC, num_subcores=NTPC)
    return pl.kernel(
        _sc_body,
        out_type=jax.ShapeDtypeStruct((NSC, NR, F), jnp.float32),
        mesh=mesh,
        scratch_types=[
            pltpu.VMEM((NCHUNK, CHUNK), jnp.int32),
            pltpu.VMEM((NCHUNK, CHUNK), jnp.int32),
            pltpu.VMEM((CHUNK, F), jnp.float32),
            pltpu.SemaphoreType.DMA,
            pltpu.VMEM_SHARED((NR, F), jnp.float32),
        ],
        compiler_params=pltpu.CompilerParams(use_tc_tiling_on_sc=False),
    )


def _sc_agg(u, src3, dst3, zeros):
    return _get_sc_agg()(u, src3, dst3, zeros)


def _t1_body(da_ref, db_ref, xp_ref, u1_ref, dis_ref):
    deg = da_ref[...] + db_ref[...] + 1.0
    dis = lax.rsqrt(deg)
    dis_ref[...] = dis
    u1_ref[...] = xp_ref[...] * dis


_t1 = pl.pallas_call(
    _t1_body,
    out_shape=(
        jax.ShapeDtypeStruct((NRR, 128), jnp.float32),
        jax.ShapeDtypeStruct((NRR, 128), jnp.float32),
    ),
)


BT2 = 1024


def _t2_body(da_ref, db_ref, u1_ref, dis_ref, w1_ref, b1_ref, w2_ref, u2_ref):
    y = dis_ref[...] * (da_ref[...] + db_ref[...] + u1_ref[...])
    h = jnp.dot(y, w1_ref[...], preferred_element_type=jnp.float32) + b1_ref[...]
    h = jnp.maximum(h, 0.0)
    z = jnp.dot(h, w2_ref[...], preferred_element_type=jnp.float32)
    u2_ref[...] = dis_ref[...] * z


_t2 = pl.pallas_call(
    _t2_body,
    grid=(NR // BT2,),
    in_specs=[
        pl.BlockSpec((BT2, F), lambda i: (i, 0)),
        pl.BlockSpec((BT2, F), lambda i: (i, 0)),
        pl.BlockSpec((BT2, F), lambda i: (i, 0)),
        pl.BlockSpec((BT2, F), lambda i: (i, 0)),
        pl.BlockSpec((F, 1024), lambda i: (0, 0)),
        pl.BlockSpec((1, 1024), lambda i: (0, 0)),
        pl.BlockSpec((1024, F), lambda i: (0, 0)),
    ],
    out_specs=pl.BlockSpec((BT2, F), lambda i: (i, 0)),
    out_shape=jax.ShapeDtypeStruct((NR, F), jnp.float32),
)


def _t3_body(da_ref, db_ref, u2_ref, dis_ref, b2_ref, out_ref):
    out_ref[...] = dis_ref[...] * (da_ref[...] + db_ref[...] + u2_ref[...]) + b2_ref[...]


_t3 = pl.pallas_call(
    _t3_body,
    out_shape=jax.ShapeDtypeStruct((NRR, 128), jnp.float32),
)


def kernel(e, pt, eta, phi, m, edge_index, W1, b1, W2, b2):
    f32 = jnp.float32
    x = jnp.concatenate([e, pt, eta, phi, m], axis=1).astype(f32)
    xp = jnp.zeros((NR, F), f32).at[:N, :5].set(x)
    ei = edge_index.astype(jnp.int32)
    pad = jnp.full((EP - E,), TRASH, jnp.int32)
    src3 = jnp.concatenate([ei[0], pad]).reshape(NTILES, NCHUNK, CHUNK)
    dst3 = jnp.concatenate([ei[1], pad]).reshape(NTILES, NCHUNK, CHUNK)
    zeros = jnp.zeros((NR, F), f32)
    ones = jnp.ones((NR, F), f32)
    W1p = jnp.zeros((F, 1024), f32).at[:5].set(W1.astype(f32))
    W2p = jnp.zeros((1024, F), f32).at[:, :3].set(W2.astype(f32))
    b2t = jnp.tile(jnp.zeros((F,), f32).at[:3].set(b2.astype(f32)), 128 // F)[None]

    d0 = _sc_agg(ones, src3, dst3, zeros)
    u1r, disr = _t1(d0[0].reshape(NRR, 128), d0[1].reshape(NRR, 128),
                    xp.reshape(NRR, 128))
    u1 = u1r.reshape(NR, F)
    d1 = _sc_agg(u1, src3, dst3, zeros)
    u2 = _t2(d1[0], d1[1], u1, disr.reshape(NR, F), W1p,
             b1.astype(f32)[None], W2p)
    d2 = _sc_agg(u2, src3, dst3, zeros)
    outr = _t3(d2[0].reshape(NRR, 128), d2[1].reshape(NRR, 128),
               u2.reshape(NRR, 128), disr, b2t)
    return outr.reshape(NR, F)[:N, :3]
